# Initial kernel scaffold; baseline (speedup 1.0000x reference)
import jax
import jax.numpy as jnp
from jax.experimental import pallas as pl


def kernel(x, edge_index, _batch, batch_ptr, params):
    raise NotImplementedError("write your pallas kernel here")


# speedup vs baseline: 5.1918x; 5.1918x over previous
import functools

import jax
import jax.numpy as jnp
from jax import lax
from jax.experimental import pallas as pl
from jax.experimental.pallas import tpu as pltpu
from jax.experimental.pallas import tpu_sc as plsc

N = 10000
D = 128
DH = 64
G = 16
G2 = 128
NC = 2
NS = 16
NW = NC * NS
CH = 128
N_PAD = 10240
RPS = N_PAD // NS
BLK = 1000
NB = N // BLK
EPS = 1e-5

_MESH = dict(core_axis_name="c", subcore_axis_name="s")


def _sc_scatter_rows(g0, g1, src_slab, dst_slab, zeros_pad):
    nch = src_slab.shape[1]

    @functools.partial(
        pl.kernel,
        out_type=jax.ShapeDtypeStruct((NC, 2, N_PAD, DH), jnp.float32),
        mesh=plsc.VectorSubcoreMesh(**_MESH),
        compiler_params=pltpu.CompilerParams(use_tc_tiling_on_sc=False),
        scratch_types=[
            pltpu.VMEM((nch, CH), jnp.int32),
            pltpu.VMEM((nch, CH), jnp.int32),
            pltpu.VMEM((CH, DH), jnp.float32),
            pltpu.VMEM((CH, DH), jnp.float32),
            pltpu.VMEM_SHARED((N_PAD, DH), jnp.float32),
            pltpu.SemaphoreType.DMA,
            pltpu.SemaphoreType.DMA,
        ],
    )
    def k(g0_hbm, g1_hbm, src_hbm, dst_hbm, z_hbm, out_hbm,
          src_v, dst_v, buf_a, buf_b, acc, sem_a, sem_b):
        c = lax.axis_index("c")
        s = lax.axis_index("s")
        wid = c * NS + s
        row0 = s * RPS
        pltpu.sync_copy(src_hbm.at[wid], src_v)
        pltpu.sync_copy(dst_hbm.at[wid], dst_v)

        for h, g_hbm in enumerate((g0_hbm, g1_hbm)):
            pltpu.sync_copy(z_hbm.at[pl.ds(row0, RPS)],
                            acc.at[pl.ds(row0, RPS)])
            plsc.subcore_barrier()

            pltpu.async_copy(g_hbm.at[src_v.at[0]], buf_a, sem_a).wait()

            def pair_body(jj, carry):
                j0 = jj * 2
                nxt = pltpu.async_copy(g_hbm.at[src_v.at[j0 + 1]], buf_b,
                                       sem_b)
                pltpu.sync_copy(buf_a, acc.at[dst_v.at[j0]], add=True)
                nxt.wait()
                j2 = lax.rem(j0 + 2, nch)
                nxt2 = pltpu.async_copy(g_hbm.at[src_v.at[j2]], buf_a, sem_a)
                pltpu.sync_copy(buf_b, acc.at[dst_v.at[j0 + 1]], add=True)
                nxt2.wait()
                return carry

            lax.fori_loop(0, nch // 2, pair_body, 0)
            plsc.subcore_barrier()
            pltpu.sync_copy(acc.at[pl.ds(row0, RPS)],
                            out_hbm.at[c, h, pl.ds(row0, RPS)])

    return k(g0, g1, src_slab, dst_slab, zeros_pad)


def _sc_degree(dst_slab, zeros_row):
    nch = dst_slab.shape[1]

    @functools.partial(
        pl.kernel,
        out_type=jax.ShapeDtypeStruct((NC, N_PAD), jnp.float32),
        mesh=plsc.VectorSubcoreMesh(**_MESH),
        compiler_params=pltpu.CompilerParams(use_tc_tiling_on_sc=False),
        scratch_types=[
            pltpu.VMEM((nch, CH), jnp.int32),
            pltpu.VMEM((CH,), jnp.float32),
            pltpu.VMEM_SHARED((N_PAD,), jnp.float32),
        ],
    )
    def k(dst_hbm, z_hbm, out_hbm, dst_v, ones_v, acc):
        c = lax.axis_index("c")
        s = lax.axis_index("s")
        wid = c * NS + s
        row0 = s * RPS

        def fill(i, carry):
            ones_v[pl.ds(i * 16, 16)] = jnp.ones((16,), jnp.float32)
            return carry

        lax.fori_loop(0, CH // 16, fill, 0)
        pltpu.sync_copy(z_hbm.at[pl.ds(row0, RPS)], acc.at[pl.ds(row0, RPS)])
        pltpu.sync_copy(dst_hbm.at[wid], dst_v)
        plsc.subcore_barrier()

        def body(j, carry):
            pltpu.sync_copy(ones_v, acc.at[dst_v.at[j]], add=True)
            return carry

        lax.fori_loop(0, nch, body, 0)
        plsc.subcore_barrier()
        pltpu.sync_copy(acc.at[pl.ds(row0, RPS)],
                        out_hbm.at[c, pl.ds(row0, RPS)])

    return k(dst_slab, zeros_row)


def _row_spec(width):
    return pl.BlockSpec((BLK, width), lambda i: (i, 0))


def _full_spec(shape):
    nd = len(shape)
    return pl.BlockSpec(shape, lambda i: (0,) * nd)


_P_SPEC = pl.BlockSpec((4, BLK, DH), lambda i: (0, i, 0))


def _gsum(p_ref, g0_ref, g1_ref):
    return jnp.concatenate(
        [p_ref[0] + p_ref[2] + g0_ref[...],
         p_ref[1] + p_ref[3] + g1_ref[...]], axis=1)


def _tc_prep(degp, x, w_in):
    def body(deg_ref, x_ref, w_ref, g0_ref, g1_ref, dinv_ref):
        d = deg_ref[0] + deg_ref[1] + 1.0
        dinv = lax.rsqrt(d)
        h = jnp.dot(x_ref[...], w_ref[...], preferred_element_type=jnp.float32)
        g = dinv * h
        g0_ref[...] = g[:, :DH]
        g1_ref[...] = g[:, DH:]
        dinv_ref[...] = dinv

    return pl.pallas_call(
        body,
        grid=(NB,),
        in_specs=[
            pl.BlockSpec((2, BLK, 1), lambda i: (0, i, 0)),
            _row_spec(D),
            _full_spec((D, D)),
        ],
        out_specs=[_row_spec(DH), _row_spec(DH), _row_spec(1)],
        out_shape=[
            jax.ShapeDtypeStruct((N, DH), jnp.float32),
            jax.ShapeDtypeStruct((N, DH), jnp.float32),
            jax.ShapeDtypeStruct((N, 1), jnp.float32),
        ],
    )(degp, x, w_in)


def _tc_zstats(p, g0, g1, dinv, b):
    def body(p_ref, g0_ref, g1_ref, dinv_ref, b_ref, z_ref, s_ref):
        i = pl.program_id(0)
        z = dinv_ref[...] * _gsum(p_ref, g0_ref, g1_ref) + b_ref[...]
        z_ref[...] = z

        @pl.when(i == 0)
        def _():
            s_ref[...] = jnp.zeros_like(s_ref)

        s_ref[0:1, :] += jnp.sum(z, axis=0, keepdims=True)
        s_ref[1:2, :] += jnp.sum(z * z, axis=0, keepdims=True)

    return pl.pallas_call(
        body,
        grid=(NB,),
        in_specs=[
            _P_SPEC,
            _row_spec(DH),
            _row_spec(DH),
            _row_spec(1),
            _full_spec((1, D)),
        ],
        out_specs=[_row_spec(D), _full_spec((8, D))],
        out_shape=[
            jax.ShapeDtypeStruct((N, D), jnp.float32),
            jax.ShapeDtypeStruct((8, D), jnp.float32),
        ],
    )(p, g0, g1, dinv, b)


def _tc_apply(z, stats, dinv, gamma, beta, w_next, pool=None):
    has_pool = pool is not None

    def body(*refs):
        if has_pool:
            (z_ref, s_ref, dinv_ref, gamma_ref, beta_ref, wn_ref,
             pw_ref, pb_ref, g0_ref, g1_ref) = refs
        else:
            (z_ref, s_ref, dinv_ref, gamma_ref, beta_ref, wn_ref,
             g0_ref, g1_ref) = refs
        mu = s_ref[0:1, :] * (1.0 / N)
        var = s_ref[1:2, :] * (1.0 / N) - mu * mu
        a = ((z_ref[...] - mu) * lax.rsqrt(var + EPS) * gamma_ref[...]
             + beta_ref[...])
        a = a * jax.nn.sigmoid(a)
        if has_pool:
            score = jax.nn.sigmoid(
                jnp.dot(a, pw_ref[...], preferred_element_type=jnp.float32)
                + pb_ref[...])
            a = a * score
        h = jnp.dot(a, wn_ref[...], preferred_element_type=jnp.float32)
        g = dinv_ref[...] * h
        g0_ref[...] = g[:, :DH]
        g1_ref[...] = g[:, DH:]

    in_specs = [
        _row_spec(D),
        _full_spec((8, D)),
        _row_spec(1),
        _full_spec((1, D)),
        _full_spec((1, D)),
        _full_spec((D, D)),
    ]
    args = [z, stats, dinv, gamma, beta, w_next]
    if has_pool:
        in_specs += [_full_spec((D, 1)), _full_spec((1, 1))]
        args += [pool['w'], pool['b'].reshape(1, 1)]
    return pl.pallas_call(
        body,
        grid=(NB,),
        in_specs=in_specs,
        out_specs=[_row_spec(DH), _row_spec(DH)],
        out_shape=[
            jax.ShapeDtypeStruct((N, DH), jnp.float32),
            jax.ShapeDtypeStruct((N, DH), jnp.float32),
        ],
    )(*args)


def _tc_final(p, g0, g1, dinv, b, batch_slab, lin1, lin2):
    def body(p_ref, g0_ref, g1_ref, dinv_ref, b_ref, br_ref,
             w1_ref, b1_ref, w2_ref, b2_ref, out_ref, acc, cnt):
        i = pl.program_id(0)

        @pl.when(i == 0)
        def _():
            acc[...] = jnp.zeros_like(acc)
            cnt[...] = jnp.zeros_like(cnt)

        z = dinv_ref[...] * _gsum(p_ref, g0_ref, g1_ref) + b_ref[...]
        xr = jnp.maximum(z, 0.0)
        ids = br_ref[0]
        gid = lax.broadcasted_iota(jnp.int32, (G2, 1), 0)
        oh = (ids == gid).astype(jnp.float32)
        acc[...] += jnp.dot(oh, xr, preferred_element_type=jnp.float32)
        cnt[...] += jnp.sum(oh, axis=1, keepdims=True)

        @pl.when(i == NB - 1)
        def _():
            mean = acc[...] / jnp.maximum(cnt[...], 1.0)
            y = jnp.dot(mean, w1_ref[...],
                        preferred_element_type=jnp.float32) + b1_ref[...]
            y = jnp.maximum(y, 0.0)
            y = jnp.dot(y, w2_ref[...],
                        preferred_element_type=jnp.float32) + b2_ref[...]
            m = jnp.max(y, axis=1, keepdims=True)
            ls = y - (m + jnp.log(jnp.sum(jnp.exp(y - m), axis=1,
                                          keepdims=True)))
            out_ref[...] = ls[0:G, :]

    return pl.pallas_call(
        body,
        grid=(NB,),
        in_specs=[
            _P_SPEC,
            _row_spec(DH),
            _row_spec(DH),
            _row_spec(1),
            _full_spec((1, D)),
            pl.BlockSpec((1, 1, BLK), lambda i: (i, 0, 0)),
            _full_spec((D, D)),
            _full_spec((1, D)),
            _full_spec((D, D)),
            _full_spec((1, D)),
        ],
        out_specs=_full_spec((G, D)),
        out_shape=jax.ShapeDtypeStruct((G, D), jnp.float32),
        scratch_shapes=[
            pltpu.VMEM((G2, D), jnp.float32),
            pltpu.VMEM((G2, 1), jnp.float32),
        ],
    )(p, g0, g1, dinv, b, batch_slab, lin1['W'], lin1['b'].reshape(1, D),
      lin2['W'], lin2['b'].reshape(1, D))


def kernel(x, edge_index, _batch, batch_ptr, params):
    e = edge_index.shape[1]
    epw = -(-e // NW)
    nch = -(-epw // CH)
    nch += nch % 2
    e_pad = NW * nch * CH

    src = jnp.concatenate(
        [edge_index[0], jnp.zeros((e_pad - e,), jnp.int32)]).reshape(NW, nch, CH)
    dst = jnp.concatenate(
        [edge_index[1], jnp.full((e_pad - e,), N, jnp.int32)]).reshape(NW, nch, CH)

    zeros_pad = jnp.zeros((N_PAD, DH), jnp.float32)
    zeros_row = jnp.zeros((N_PAD,), jnp.float32)
    batch_slab = _batch.reshape(NB, 1, BLK)

    degp = _sc_degree(dst, zeros_row)
    g0, g1, dinv = _tc_prep(degp.reshape(2, N_PAD, 1), x,
                            params['input_block']['W'])

    ib = params['input_block']
    rb = params['res_blocks']
    pools = params['pools']
    conv_params = [ib, rb[0], rb[0], rb[2], rb[2], rb[0], rb[0]]
    next_w = [rb[0]['W'], rb[0]['W'], rb[2]['W'], rb[2]['W'], rb[0]['W'],
              rb[0]['W'], params['conv3']['W']]
    pool_after = [None, None, pools[0], None, pools[1], None, pools[2]]

    for i in range(7):
        p = _sc_scatter_rows(g0, g1, src, dst, zeros_pad)
        p = p.reshape(NC * 2, N_PAD, DH)
        cp = conv_params[i]
        z, stats = _tc_zstats(p, g0, g1, dinv, cp['b'].reshape(1, D))
        g0, g1 = _tc_apply(z, stats, dinv, cp['gamma'].reshape(1, D),
                           cp['beta'].reshape(1, D), next_w[i], pool_after[i])

    p = _sc_scatter_rows(g0, g1, src, dst, zeros_pad)
    p = p.reshape(NC * 2, N_PAD, DH)
    out = _tc_final(p, g0, g1, dinv, params['conv3']['b'].reshape(1, D),
                    batch_slab, params['lin1'], params['lin2'])
    return (out, jnp.array(0.0, dtype=jnp.float32))


# speedup vs baseline: 5.6471x; 1.0877x over previous
import functools

import jax
import jax.numpy as jnp
from jax import lax
from jax.experimental import pallas as pl
from jax.experimental.pallas import tpu as pltpu
from jax.experimental.pallas import tpu_sc as plsc

N = 10000
D = 128
DH = 64
G = 16
G2 = 128
NC = 2
NS = 16
NW = NC * NS
CH = 128
GRP = 4
N_PAD = 10240
RPS = N_PAD // NS
BLK = 1000
NB = N // BLK
EPS = 1e-5

_MESH = dict(core_axis_name="c", subcore_axis_name="s")


def _sc_scatter_rows(g0, g1, src_slab, dst_slab, zeros_pad):
    nch = src_slab.shape[1]

    @functools.partial(
        pl.kernel,
        out_type=jax.ShapeDtypeStruct((NC, 2, N_PAD, DH), jnp.float32),
        mesh=plsc.VectorSubcoreMesh(**_MESH),
        compiler_params=pltpu.CompilerParams(use_tc_tiling_on_sc=False),
        scratch_types=[
            pltpu.VMEM((nch, CH), jnp.int32),
            pltpu.VMEM((nch, CH), jnp.int32),
            [pltpu.VMEM((CH, DH), jnp.float32) for _ in range(GRP)],
            [pltpu.VMEM((CH, DH), jnp.float32) for _ in range(GRP)],
            pltpu.VMEM_SHARED((N_PAD, DH), jnp.float32),
            pltpu.SemaphoreType.DMA,
            pltpu.SemaphoreType.DMA,
            pltpu.SemaphoreType.DMA,
            pltpu.SemaphoreType.DMA,
        ],
    )
    def k(g0_hbm, g1_hbm, src_hbm, dst_hbm, z_hbm, out_hbm,
          src_v, dst_v, bufs_a, bufs_b, acc, sga, sgb, ssa, ssb):
        c = lax.axis_index("c")
        s = lax.axis_index("s")
        wid = c * NS + s
        row0 = s * RPS
        pltpu.sync_copy(src_hbm.at[wid], src_v)
        pltpu.sync_copy(dst_hbm.at[wid], dst_v)
        npairs = nch // (2 * GRP)

        def fire_gathers(g_hbm, grp, bufs, sem):
            waits = []
            for b in range(GRP):
                j = lax.rem(grp * GRP + b, nch)
                waits.append(
                    pltpu.async_copy(g_hbm.at[src_v.at[j]], bufs[b], sem))
            return waits

        def fire_scatters(grp, bufs, sem):
            waits = []
            for b in range(GRP):
                j = grp * GRP + b
                waits.append(
                    pltpu.async_copy(bufs[b], acc.at[dst_v.at[j]], sem,
                                     add=True))
            return waits

        for h, g_hbm in enumerate((g0_hbm, g1_hbm)):
            pltpu.sync_copy(z_hbm.at[pl.ds(row0, RPS)],
                            acc.at[pl.ds(row0, RPS)])
            plsc.subcore_barrier()

            fire_gathers(g_hbm, 0, bufs_a, sga)

            def pipe(jj, carry):
                ga = jj * 2
                gb = ga + 1
                for b in range(GRP):
                    pltpu.make_async_copy(g_hbm.at[src_v.at[0]],
                                          bufs_a[b], sga).wait()
                fire_gathers(g_hbm, gb, bufs_b, sgb)
                sw_a = fire_scatters(ga, bufs_a, ssa)
                for w in sw_a:
                    w.wait()
                fire_gathers(g_hbm, ga + 2, bufs_a, sga)
                for b in range(GRP):
                    pltpu.make_async_copy(g_hbm.at[src_v.at[0]],
                                          bufs_b[b], sgb).wait()
                sw_b = fire_scatters(gb, bufs_b, ssb)
                for w in sw_b:
                    w.wait()
                return carry

            lax.fori_loop(0, npairs, pipe, 0)
            for b in range(GRP):
                pltpu.make_async_copy(g_hbm.at[src_v.at[0]],
                                      bufs_a[b], sga).wait()
            plsc.subcore_barrier()
            pltpu.sync_copy(acc.at[pl.ds(row0, RPS)],
                            out_hbm.at[c, h, pl.ds(row0, RPS)])

    return k(g0, g1, src_slab, dst_slab, zeros_pad)


def _sc_degree(dst_slab, zeros_row):
    nch = dst_slab.shape[1]

    @functools.partial(
        pl.kernel,
        out_type=jax.ShapeDtypeStruct((NC, N_PAD), jnp.float32),
        mesh=plsc.VectorSubcoreMesh(**_MESH),
        compiler_params=pltpu.CompilerParams(use_tc_tiling_on_sc=False),
        scratch_types=[
            pltpu.VMEM((nch, CH), jnp.int32),
            pltpu.VMEM((CH,), jnp.float32),
            pltpu.VMEM_SHARED((N_PAD,), jnp.float32),
        ],
    )
    def k(dst_hbm, z_hbm, out_hbm, dst_v, ones_v, acc):
        c = lax.axis_index("c")
        s = lax.axis_index("s")
        wid = c * NS + s
        row0 = s * RPS

        def fill(i, carry):
            ones_v[pl.ds(i * 16, 16)] = jnp.ones((16,), jnp.float32)
            return carry

        lax.fori_loop(0, CH // 16, fill, 0)
        pltpu.sync_copy(z_hbm.at[pl.ds(row0, RPS)], acc.at[pl.ds(row0, RPS)])
        pltpu.sync_copy(dst_hbm.at[wid], dst_v)
        plsc.subcore_barrier()

        def body(j, carry):
            pltpu.sync_copy(ones_v, acc.at[dst_v.at[j]], add=True)
            return carry

        lax.fori_loop(0, nch, body, 0)
        plsc.subcore_barrier()
        pltpu.sync_copy(acc.at[pl.ds(row0, RPS)],
                        out_hbm.at[c, pl.ds(row0, RPS)])

    return k(dst_slab, zeros_row)


def _row_spec(width):
    return pl.BlockSpec((BLK, width), lambda i: (i, 0))


def _full_spec(shape):
    nd = len(shape)
    return pl.BlockSpec(shape, lambda i: (0,) * nd)


_P_SPEC = pl.BlockSpec((4, BLK, DH), lambda i: (0, i, 0))


def _gsum(p_ref, g0_ref, g1_ref):
    return jnp.concatenate(
        [p_ref[0] + p_ref[2] + g0_ref[...],
         p_ref[1] + p_ref[3] + g1_ref[...]], axis=1)


def _tc_prep(degp, x, w_in):
    def body(deg_ref, x_ref, w_ref, g0_ref, g1_ref, dinv_ref):
        d = deg_ref[0] + deg_ref[1] + 1.0
        dinv = lax.rsqrt(d)
        h = jnp.dot(x_ref[...], w_ref[...], preferred_element_type=jnp.float32)
        g = dinv * h
        g0_ref[...] = g[:, :DH]
        g1_ref[...] = g[:, DH:]
        dinv_ref[...] = dinv

    return pl.pallas_call(
        body,
        grid=(NB,),
        in_specs=[
            pl.BlockSpec((2, BLK, 1), lambda i: (0, i, 0)),
            _row_spec(D),
            _full_spec((D, D)),
        ],
        out_specs=[_row_spec(DH), _row_spec(DH), _row_spec(1)],
        out_shape=[
            jax.ShapeDtypeStruct((N, DH), jnp.float32),
            jax.ShapeDtypeStruct((N, DH), jnp.float32),
            jax.ShapeDtypeStruct((N, 1), jnp.float32),
        ],
    )(degp, x, w_in)


def _tc_zstats(p, g0, g1, dinv, b):
    def body(p_ref, g0_ref, g1_ref, dinv_ref, b_ref, z_ref, s_ref):
        i = pl.program_id(0)
        z = dinv_ref[...] * _gsum(p_ref, g0_ref, g1_ref) + b_ref[...]
        z_ref[...] = z

        @pl.when(i == 0)
        def _():
            s_ref[...] = jnp.zeros_like(s_ref)

        s_ref[0:1, :] += jnp.sum(z, axis=0, keepdims=True)
        s_ref[1:2, :] += jnp.sum(z * z, axis=0, keepdims=True)

    return pl.pallas_call(
        body,
        grid=(NB,),
        in_specs=[
            _P_SPEC,
            _row_spec(DH),
            _row_spec(DH),
            _row_spec(1),
            _full_spec((1, D)),
        ],
        out_specs=[_row_spec(D), _full_spec((8, D))],
        out_shape=[
            jax.ShapeDtypeStruct((N, D), jnp.float32),
            jax.ShapeDtypeStruct((8, D), jnp.float32),
        ],
    )(p, g0, g1, dinv, b)


def _tc_apply(z, stats, dinv, gamma, beta, w_next, pool=None):
    has_pool = pool is not None

    def body(*refs):
        if has_pool:
            (z_ref, s_ref, dinv_ref, gamma_ref, beta_ref, wn_ref,
             pw_ref, pb_ref, g0_ref, g1_ref) = refs
        else:
            (z_ref, s_ref, dinv_ref, gamma_ref, beta_ref, wn_ref,
             g0_ref, g1_ref) = refs
        mu = s_ref[0:1, :] * (1.0 / N)
        var = s_ref[1:2, :] * (1.0 / N) - mu * mu
        a = ((z_ref[...] - mu) * lax.rsqrt(var + EPS) * gamma_ref[...]
             + beta_ref[...])
        a = a * jax.nn.sigmoid(a)
        if has_pool:
            score = jax.nn.sigmoid(
                jnp.dot(a, pw_ref[...], preferred_element_type=jnp.float32)
                + pb_ref[...])
            a = a * score
        h = jnp.dot(a, wn_ref[...], preferred_element_type=jnp.float32)
        g = dinv_ref[...] * h
        g0_ref[...] = g[:, :DH]
        g1_ref[...] = g[:, DH:]

    in_specs = [
        _row_spec(D),
        _full_spec((8, D)),
        _row_spec(1),
        _full_spec((1, D)),
        _full_spec((1, D)),
        _full_spec((D, D)),
    ]
    args = [z, stats, dinv, gamma, beta, w_next]
    if has_pool:
        in_specs += [_full_spec((D, 1)), _full_spec((1, 1))]
        args += [pool['w'], pool['b'].reshape(1, 1)]
    return pl.pallas_call(
        body,
        grid=(NB,),
        in_specs=in_specs,
        out_specs=[_row_spec(DH), _row_spec(DH)],
        out_shape=[
            jax.ShapeDtypeStruct((N, DH), jnp.float32),
            jax.ShapeDtypeStruct((N, DH), jnp.float32),
        ],
    )(*args)


def _tc_final(p, g0, g1, dinv, b, batch_slab, lin1, lin2):
    def body(p_ref, g0_ref, g1_ref, dinv_ref, b_ref, br_ref,
             w1_ref, b1_ref, w2_ref, b2_ref, out_ref, acc, cnt):
        i = pl.program_id(0)

        @pl.when(i == 0)
        def _():
            acc[...] = jnp.zeros_like(acc)
            cnt[...] = jnp.zeros_like(cnt)

        z = dinv_ref[...] * _gsum(p_ref, g0_ref, g1_ref) + b_ref[...]
        xr = jnp.maximum(z, 0.0)
        ids = br_ref[0]
        gid = lax.broadcasted_iota(jnp.int32, (G2, 1), 0)
        oh = (ids == gid).astype(jnp.float32)
        acc[...] += jnp.dot(oh, xr, preferred_element_type=jnp.float32)
        cnt[...] += jnp.sum(oh, axis=1, keepdims=True)

        @pl.when(i == NB - 1)
        def _():
            mean = acc[...] / jnp.maximum(cnt[...], 1.0)
            y = jnp.dot(mean, w1_ref[...],
                        preferred_element_type=jnp.float32) + b1_ref[...]
            y = jnp.maximum(y, 0.0)
            y = jnp.dot(y, w2_ref[...],
                        preferred_element_type=jnp.float32) + b2_ref[...]
            m = jnp.max(y, axis=1, keepdims=True)
            ls = y - (m + jnp.log(jnp.sum(jnp.exp(y - m), axis=1,
                                          keepdims=True)))
            out_ref[...] = ls[0:G, :]

    return pl.pallas_call(
        body,
        grid=(NB,),
        in_specs=[
            _P_SPEC,
            _row_spec(DH),
            _row_spec(DH),
            _row_spec(1),
            _full_spec((1, D)),
            pl.BlockSpec((1, 1, BLK), lambda i: (i, 0, 0)),
            _full_spec((D, D)),
            _full_spec((1, D)),
            _full_spec((D, D)),
            _full_spec((1, D)),
        ],
        out_specs=_full_spec((G, D)),
        out_shape=jax.ShapeDtypeStruct((G, D), jnp.float32),
        scratch_shapes=[
            pltpu.VMEM((G2, D), jnp.float32),
            pltpu.VMEM((G2, 1), jnp.float32),
        ],
    )(p, g0, g1, dinv, b, batch_slab, lin1['W'], lin1['b'].reshape(1, D),
      lin2['W'], lin2['b'].reshape(1, D))


def kernel(x, edge_index, _batch, batch_ptr, params):
    e = edge_index.shape[1]
    epw = -(-e // NW)
    nch = -(-epw // CH)
    nch = -(-nch // (2 * GRP)) * (2 * GRP)
    e_pad = NW * nch * CH

    src = jnp.concatenate(
        [edge_index[0], jnp.zeros((e_pad - e,), jnp.int32)]).reshape(NW, nch, CH)
    dst = jnp.concatenate(
        [edge_index[1], jnp.full((e_pad - e,), N, jnp.int32)]).reshape(NW, nch, CH)

    zeros_pad = jnp.zeros((N_PAD, DH), jnp.float32)
    zeros_row = jnp.zeros((N_PAD,), jnp.float32)
    batch_slab = _batch.reshape(NB, 1, BLK)

    degp = _sc_degree(dst, zeros_row)
    g0, g1, dinv = _tc_prep(degp.reshape(2, N_PAD, 1), x,
                            params['input_block']['W'])

    ib = params['input_block']
    rb = params['res_blocks']
    pools = params['pools']
    conv_params = [ib, rb[0], rb[0], rb[2], rb[2], rb[0], rb[0]]
    next_w = [rb[0]['W'], rb[0]['W'], rb[2]['W'], rb[2]['W'], rb[0]['W'],
              rb[0]['W'], params['conv3']['W']]
    pool_after = [None, None, pools[0], None, pools[1], None, pools[2]]

    for i in range(7):
        p = _sc_scatter_rows(g0, g1, src, dst, zeros_pad)
        p = p.reshape(NC * 2, N_PAD, DH)
        cp = conv_params[i]
        z, stats = _tc_zstats(p, g0, g1, dinv, cp['b'].reshape(1, D))
        g0, g1 = _tc_apply(z, stats, dinv, cp['gamma'].reshape(1, D),
                           cp['beta'].reshape(1, D), next_w[i], pool_after[i])

    p = _sc_scatter_rows(g0, g1, src, dst, zeros_pad)
    p = p.reshape(NC * 2, N_PAD, DH)
    out = _tc_final(p, g0, g1, dinv, params['conv3']['b'].reshape(1, D),
                    batch_slab, params['lin1'], params['lin2'])
    return (out, jnp.array(0.0, dtype=jnp.float32))
